# baseline (speedup 1.0000x reference)
import functools

import jax
import jax.numpy as jnp
from jax import lax
from jax.experimental import pallas as pl
from jax.experimental.pallas import tpu as pltpu
from jax.experimental.pallas import tpu_sc as plsc

N_NODES = 200
WINDOW = 200
TOTAL = N_NODES * WINDOW
D = 128
T = 2
E = 100000
H1 = 1024

E_PAD = 100352
CHUNK = 128
EPT = E_PAD // 16
NCH = EPT // CHUNK
QC = 16
SEG_PT = TOTAL // 16


def _sc_segment_sum(m, dst2d, zrows):
    mesh = plsc.VectorSubcoreMesh(core_axis_name="c", subcore_axis_name="s")

    @functools.partial(
        pl.kernel, mesh=mesh,
        out_type=jax.ShapeDtypeStruct((TOTAL, D), jnp.float32),
        compiler_params=pltpu.CompilerParams(use_tc_tiling_on_sc=False),
        scratch_types=[
            pltpu.VMEM((NCH, CHUNK), jnp.int32),
            pltpu.VMEM((CHUNK, QC), jnp.float32),
            pltpu.VMEM((SEG_PT, QC), jnp.float32),
            pltpu.VMEM_SHARED((TOTAL, QC), jnp.float32),
        ],
    )
    def seg_sum(m_hbm, dst_hbm, z_hbm, out_hbm, idx_v, rows_v, stage_v, acc_sh):
        c = lax.axis_index("c")
        s = lax.axis_index("s")
        pltpu.sync_copy(dst_hbm.at[pl.ds(s * NCH, NCH)], idx_v)
        for r in range(4):
            q = c * 4 + r
            col = pl.multiple_of(q * QC, QC)
            pltpu.sync_copy(z_hbm, stage_v)
            pltpu.sync_copy(stage_v, acc_sh.at[pl.ds(s * SEG_PT, SEG_PT)])
            plsc.subcore_barrier()

            def body(j, _):
                pltpu.sync_copy(
                    m_hbm.at[pl.ds(s * EPT + j * CHUNK, CHUNK), pl.ds(col, QC)],
                    rows_v)
                pltpu.sync_copy(rows_v, acc_sh.at[idx_v.at[j]], add=True)
                return _

            lax.fori_loop(0, NCH, body, None)
            plsc.subcore_barrier()
            pltpu.sync_copy(acc_sh.at[pl.ds(s * SEG_PT, SEG_PT)], stage_v)
            pltpu.sync_copy(
                stage_v,
                out_hbm.at[pl.ds(s * SEG_PT, SEG_PT), pl.ds(col, QC)])
            if r < 3:
                plsc.subcore_barrier()

    return seg_sum(m, dst2d, zrows)


def _msg_mlp(ha, hb, W1a, W1b, b1, W2, b2, *, block, n_valid=None):
    n = ha.shape[0]
    assert n % block == 0
    grid = n // block

    def kern(ha_ref, hb_ref, w1a_ref, w1b_ref, b1_ref, w2_ref, b2_ref, o_ref):
        acc = jnp.dot(ha_ref[...].astype(jnp.bfloat16), w1a_ref[...],
                      preferred_element_type=jnp.float32)
        acc = acc + jnp.dot(hb_ref[...].astype(jnp.bfloat16), w1b_ref[...],
                            preferred_element_type=jnp.float32)
        hid = jnp.maximum(acc + b1_ref[...], 0.0).astype(jnp.bfloat16)
        out = jnp.dot(hid, w2_ref[...], preferred_element_type=jnp.float32) + b2_ref[...]
        out = jnp.maximum(out, 0.0)
        if n_valid is not None:
            row = (lax.broadcasted_iota(jnp.int32, (block, 1), 0)
                   + pl.program_id(0) * block)
            out = jnp.where(row < n_valid, out, 0.0)
        o_ref[...] = out

    return pl.pallas_call(
        kern,
        grid=(grid,),
        in_specs=[
            pl.BlockSpec((block, D), lambda i: (i, 0)),
            pl.BlockSpec((block, D), lambda i: (i, 0)),
            pl.BlockSpec((D, H1), lambda i: (0, 0)),
            pl.BlockSpec((D, H1), lambda i: (0, 0)),
            pl.BlockSpec((1, H1), lambda i: (0, 0)),
            pl.BlockSpec((H1, D), lambda i: (0, 0)),
            pl.BlockSpec((1, D), lambda i: (0, 0)),
        ],
        out_specs=pl.BlockSpec((block, D), lambda i: (i, 0)),
        out_shape=jax.ShapeDtypeStruct((n, D), jnp.float32),
    )(ha, hb, W1a, W1b, b1, W2, b2)


def _gru(sums, counts, h, k, rk, b, *, block):
    n = h.shape[0]
    assert n % block == 0
    grid = n // block

    def kern(s_ref, c_ref, h_ref, k_ref, rk_ref, b_ref, o_ref):
        c = c_ref[...]
        x = jnp.where(c > 0.0, s_ref[...] / jnp.maximum(c, 1.0), 0.0)
        hv = h_ref[...]
        mx = jnp.dot(x.astype(jnp.bfloat16), k_ref[...],
                     preferred_element_type=jnp.float32) + b_ref[0:1, :]
        mh = jnp.dot(hv.astype(jnp.bfloat16), rk_ref[...],
                     preferred_element_type=jnp.float32) + b_ref[1:2, :]
        z = jax.nn.sigmoid(mx[:, :D] + mh[:, :D])
        r = jax.nn.sigmoid(mx[:, D:2 * D] + mh[:, D:2 * D])
        cand = jnp.tanh(mx[:, 2 * D:] + r * mh[:, 2 * D:])
        o_ref[...] = z * hv + (1.0 - z) * cand

    return pl.pallas_call(
        kern,
        grid=(grid,),
        in_specs=[
            pl.BlockSpec((block, D), lambda i: (i, 0)),
            pl.BlockSpec((block, 1), lambda i: (i, 0)),
            pl.BlockSpec((block, D), lambda i: (i, 0)),
            pl.BlockSpec((D, 3 * D), lambda i: (0, 0)),
            pl.BlockSpec((D, 3 * D), lambda i: (0, 0)),
            pl.BlockSpec((2, 3 * D), lambda i: (0, 0)),
        ],
        out_specs=pl.BlockSpec((block, D), lambda i: (i, 0)),
        out_shape=jax.ShapeDtypeStruct((n, D), jnp.float32),
    )(sums, counts, h, k, rk, b)


def _readout(x, W1, b1, W2, b2, W3p, b3p):
    def kern(x_ref, w1_ref, b1_ref, w2_ref, b2_ref, w3_ref, b3_ref, o_ref):
        a = jnp.maximum(
            jnp.dot(x_ref[...], w1_ref[...], preferred_element_type=jnp.float32)
            + b1_ref[...], 0.0)
        a = jnp.maximum(
            jnp.dot(a, w2_ref[...], preferred_element_type=jnp.float32)
            + b2_ref[...], 0.0)
        lg = jnp.dot(a, w3_ref[...], preferred_element_type=jnp.float32) + b3_ref[...]
        mx = jnp.max(lg, axis=-1, keepdims=True)
        ex = jnp.exp(lg - mx)
        o_ref[...] = ex / jnp.sum(ex, axis=-1, keepdims=True)

    return pl.pallas_call(
        kern,
        out_shape=jax.ShapeDtypeStruct((N_NODES, D), jnp.float32),
    )(x, W1, b1, W2, b2, W3p, b3p)


def kernel(int_edges, nodes, embed, msg_W1, msg_b1, msg_W2, msg_b2,
           gru_int_k, gru_int_rk, gru_int_b, gru_temp_k, gru_temp_rk,
           gru_temp_b, ro_W1, ro_b1, ro_W2, ro_b2, ro_W3, ro_b3):
    node_idx = int_edges[:, 0] * N_NODES + int_edges[:, 1]
    nbr_idx = int_edges[:, 0] * N_NODES + int_edges[:, 2]
    ipad = jnp.zeros((E_PAD - E,), jnp.int32)
    node_idx_p = jnp.concatenate([node_idx, ipad])
    nbr_idx_p = jnp.concatenate([nbr_idx, ipad])
    dst2d = nbr_idx_p.reshape(E_PAD // CHUNK, CHUNK)
    zrows = jnp.zeros((SEG_PT, QC), jnp.float32)

    h = jnp.repeat(embed[nodes], WINDOW, axis=0)

    W1a = msg_W1[:D].astype(jnp.bfloat16)
    W1b = msg_W1[D:].astype(jnp.bfloat16)
    b1r = msg_b1.reshape(1, H1)
    b2r = msg_b2.reshape(1, D)
    W2c = msg_W2.astype(jnp.bfloat16)
    gik = gru_int_k.astype(jnp.bfloat16)
    girk = gru_int_rk.astype(jnp.bfloat16)
    gtk = gru_temp_k.astype(jnp.bfloat16)
    gtrk = gru_temp_rk.astype(jnp.bfloat16)

    io = jnp.arange(TOTAL, dtype=jnp.int32)
    tcount = jnp.where((io >= N_NODES) & (io < TOTAL - N_NODES), 2.0, 1.0)[:, None]

    for _ in range(T):
        ha = jnp.take(h, node_idx_p, axis=0)
        hb = jnp.take(h, nbr_idx_p, axis=0)
        m = _msg_mlp(ha, hb, W1a, W1b, b1r, W2c, b2r, block=2048, n_valid=E)
        sums = _sc_segment_sum(m, dst2d, zrows)
        counts = jnp.zeros((TOTAL, 1), jnp.float32).at[nbr_idx].add(1.0)
        h = _gru(sums, counts, h, gik, girk, gru_int_b, block=2000)

        hs = jnp.roll(h, -N_NODES, axis=0)
        m1 = _msg_mlp(h, hs, W1a, W1b, b1r, W2c, b2r, block=2000)
        m2 = _msg_mlp(hs, h, W1a, W1b, b1r, W2c, b2r, block=2000)
        z200 = jnp.zeros((N_NODES, D), jnp.float32)
        tsum = (jnp.concatenate([z200, m1[:TOTAL - N_NODES]], axis=0)
                + jnp.concatenate([m2[:TOTAL - N_NODES], z200], axis=0))
        h = _gru(tsum, tcount, h, gtk, gtrk, gru_temp_b, block=2000)

    W3p = jnp.zeros((512, D), jnp.float32).at[:, :10].set(ro_W3)
    b3p = jnp.full((1, D), -1e30, jnp.float32).at[0, :10].set(ro_b3)
    probs = _readout(h[:N_NODES], ro_W1, ro_b1.reshape(1, H1),
                     ro_W2, ro_b2.reshape(1, 512), W3p, b3p)
    return probs[:, :10]


# speedup vs baseline: 1.0656x; 1.0656x over previous
import functools

import jax
import jax.numpy as jnp
from jax import lax
from jax.experimental import pallas as pl
from jax.experimental.pallas import tpu as pltpu
from jax.experimental.pallas import tpu_sc as plsc

N_NODES = 200
WINDOW = 200
TOTAL = N_NODES * WINDOW
D = 128
T = 2
E = 100000
H1 = 1024

E_PAD = 100352
CHUNK = 128
EPT = E_PAD // 16
NCH = EPT // CHUNK
QC = 16
SEG_PT = TOTAL // 16
BCH = 7
NB = NCH // BCH


def _sc_segment_sum(m, dst2d, zrows):
    mesh = plsc.VectorSubcoreMesh(core_axis_name="c", subcore_axis_name="s")

    @functools.partial(
        pl.kernel, mesh=mesh,
        out_type=jax.ShapeDtypeStruct((TOTAL, D), jnp.float32),
        compiler_params=pltpu.CompilerParams(use_tc_tiling_on_sc=False),
        scratch_types=[
            pltpu.VMEM((NCH, CHUNK), jnp.int32),
            pltpu.VMEM((BCH * CHUNK, QC), jnp.float32),
            pltpu.VMEM((BCH * CHUNK, QC), jnp.float32),
            pltpu.VMEM_SHARED((TOTAL, QC), jnp.float32),
            pltpu.SemaphoreType.DMA,
            pltpu.SemaphoreType.DMA,
            pltpu.SemaphoreType.DMA,
            pltpu.SemaphoreType.DMA,
        ],
    )
    def seg_sum(m_hbm, dst_hbm, z_hbm, out_hbm, idx_v, rows_a, rows_b,
                acc_sh, gs0, gs1, ss0, ss1):
        c = lax.axis_index("c")
        s = lax.axis_index("s")
        bufs = (rows_a, rows_b)
        gsems = (gs0, gs1)
        ssems = (ss0, ss1)
        pltpu.sync_copy(dst_hbm.at[pl.ds(s * NCH, NCH)], idx_v)
        for r in range(4):
            q = c * 4 + r
            col = pl.multiple_of(q * QC, QC)
            pltpu.sync_copy(z_hbm, acc_sh.at[pl.ds(s * SEG_PT, SEG_PT)])
            plsc.subcore_barrier()

            def gather(b):
                return pltpu.async_copy(
                    m_hbm.at[pl.ds(s * EPT + b * BCH * CHUNK, BCH * CHUNK),
                             pl.ds(col, QC)],
                    bufs[b % 2], gsems[b % 2])

            gh = gather(0)
            scat = {0: [], 1: []}
            for b in range(NB):
                cur = b % 2
                nxt = 1 - cur
                if b + 1 < NB:
                    for hd in scat[nxt]:
                        hd.wait()
                    scat[nxt] = []
                    gh_next = gather(b + 1)
                gh.wait()
                for k in range(BCH):
                    scat[cur].append(pltpu.async_copy(
                        bufs[cur].at[pl.ds(k * CHUNK, CHUNK)],
                        acc_sh.at[idx_v.at[b * BCH + k]],
                        ssems[cur], add=True))
                if b + 1 < NB:
                    gh = gh_next
            for side in (0, 1):
                for hd in scat[side]:
                    hd.wait()
            plsc.subcore_barrier()
            pltpu.sync_copy(
                acc_sh.at[pl.ds(s * SEG_PT, SEG_PT)],
                out_hbm.at[pl.ds(s * SEG_PT, SEG_PT), pl.ds(col, QC)])
            if r < 3:
                plsc.subcore_barrier()

    return seg_sum(m, dst2d, zrows)


def _msg_mlp(ha, hb, W1a, W1b, b1, W2, b2, *, block, n_valid=None):
    n = ha.shape[0]
    assert n % block == 0
    grid = n // block

    def kern(ha_ref, hb_ref, w1a_ref, w1b_ref, b1_ref, w2_ref, b2_ref, o_ref):
        acc = jnp.dot(ha_ref[...].astype(jnp.bfloat16), w1a_ref[...],
                      preferred_element_type=jnp.float32)
        acc = acc + jnp.dot(hb_ref[...].astype(jnp.bfloat16), w1b_ref[...],
                            preferred_element_type=jnp.float32)
        hid = jnp.maximum(acc + b1_ref[...], 0.0).astype(jnp.bfloat16)
        out = jnp.dot(hid, w2_ref[...], preferred_element_type=jnp.float32) + b2_ref[...]
        out = jnp.maximum(out, 0.0)
        if n_valid is not None:
            row = (lax.broadcasted_iota(jnp.int32, (block, 1), 0)
                   + pl.program_id(0) * block)
            out = jnp.where(row < n_valid, out, 0.0)
        o_ref[...] = out

    return pl.pallas_call(
        kern,
        grid=(grid,),
        in_specs=[
            pl.BlockSpec((block, D), lambda i: (i, 0)),
            pl.BlockSpec((block, D), lambda i: (i, 0)),
            pl.BlockSpec((D, H1), lambda i: (0, 0)),
            pl.BlockSpec((D, H1), lambda i: (0, 0)),
            pl.BlockSpec((1, H1), lambda i: (0, 0)),
            pl.BlockSpec((H1, D), lambda i: (0, 0)),
            pl.BlockSpec((1, D), lambda i: (0, 0)),
        ],
        out_specs=pl.BlockSpec((block, D), lambda i: (i, 0)),
        out_shape=jax.ShapeDtypeStruct((n, D), jnp.float32),
    )(ha, hb, W1a, W1b, b1, W2, b2)


def _gru(sums, counts, h, k, rk, b, *, block):
    n = h.shape[0]
    assert n % block == 0
    grid = n // block

    def kern(s_ref, c_ref, h_ref, k_ref, rk_ref, b_ref, o_ref):
        c = c_ref[...]
        x = jnp.where(c > 0.0, s_ref[...] / jnp.maximum(c, 1.0), 0.0)
        hv = h_ref[...]
        mx = jnp.dot(x.astype(jnp.bfloat16), k_ref[...],
                     preferred_element_type=jnp.float32) + b_ref[0:1, :]
        mh = jnp.dot(hv.astype(jnp.bfloat16), rk_ref[...],
                     preferred_element_type=jnp.float32) + b_ref[1:2, :]
        z = jax.nn.sigmoid(mx[:, :D] + mh[:, :D])
        r = jax.nn.sigmoid(mx[:, D:2 * D] + mh[:, D:2 * D])
        cand = jnp.tanh(mx[:, 2 * D:] + r * mh[:, 2 * D:])
        o_ref[...] = z * hv + (1.0 - z) * cand

    return pl.pallas_call(
        kern,
        grid=(grid,),
        in_specs=[
            pl.BlockSpec((block, D), lambda i: (i, 0)),
            pl.BlockSpec((block, 1), lambda i: (i, 0)),
            pl.BlockSpec((block, D), lambda i: (i, 0)),
            pl.BlockSpec((D, 3 * D), lambda i: (0, 0)),
            pl.BlockSpec((D, 3 * D), lambda i: (0, 0)),
            pl.BlockSpec((2, 3 * D), lambda i: (0, 0)),
        ],
        out_specs=pl.BlockSpec((block, D), lambda i: (i, 0)),
        out_shape=jax.ShapeDtypeStruct((n, D), jnp.float32),
    )(sums, counts, h, k, rk, b)


def _readout(x, W1, b1, W2, b2, W3p, b3p):
    def kern(x_ref, w1_ref, b1_ref, w2_ref, b2_ref, w3_ref, b3_ref, o_ref):
        a = jnp.maximum(
            jnp.dot(x_ref[...], w1_ref[...], preferred_element_type=jnp.float32)
            + b1_ref[...], 0.0)
        a = jnp.maximum(
            jnp.dot(a, w2_ref[...], preferred_element_type=jnp.float32)
            + b2_ref[...], 0.0)
        lg = jnp.dot(a, w3_ref[...], preferred_element_type=jnp.float32) + b3_ref[...]
        mx = jnp.max(lg, axis=-1, keepdims=True)
        ex = jnp.exp(lg - mx)
        o_ref[...] = ex / jnp.sum(ex, axis=-1, keepdims=True)

    return pl.pallas_call(
        kern,
        out_shape=jax.ShapeDtypeStruct((N_NODES, D), jnp.float32),
    )(x, W1, b1, W2, b2, W3p, b3p)


def kernel(int_edges, nodes, embed, msg_W1, msg_b1, msg_W2, msg_b2,
           gru_int_k, gru_int_rk, gru_int_b, gru_temp_k, gru_temp_rk,
           gru_temp_b, ro_W1, ro_b1, ro_W2, ro_b2, ro_W3, ro_b3):
    node_idx = int_edges[:, 0] * N_NODES + int_edges[:, 1]
    nbr_idx = int_edges[:, 0] * N_NODES + int_edges[:, 2]
    ipad = jnp.zeros((E_PAD - E,), jnp.int32)
    node_idx_p = jnp.concatenate([node_idx, ipad])
    nbr_idx_p = jnp.concatenate([nbr_idx, ipad])
    dst2d = nbr_idx_p.reshape(E_PAD // CHUNK, CHUNK)
    zrows = jnp.zeros((SEG_PT, QC), jnp.float32)

    h = jnp.repeat(embed[nodes], WINDOW, axis=0)

    W1a = msg_W1[:D].astype(jnp.bfloat16)
    W1b = msg_W1[D:].astype(jnp.bfloat16)
    b1r = msg_b1.reshape(1, H1)
    b2r = msg_b2.reshape(1, D)
    W2c = msg_W2.astype(jnp.bfloat16)
    gik = gru_int_k.astype(jnp.bfloat16)
    girk = gru_int_rk.astype(jnp.bfloat16)
    gtk = gru_temp_k.astype(jnp.bfloat16)
    gtrk = gru_temp_rk.astype(jnp.bfloat16)

    io = jnp.arange(TOTAL, dtype=jnp.int32)
    tcount = jnp.where((io >= N_NODES) & (io < TOTAL - N_NODES), 2.0, 1.0)[:, None]

    for _ in range(T):
        ha = jnp.take(h, node_idx_p, axis=0)
        hb = jnp.take(h, nbr_idx_p, axis=0)
        m = _msg_mlp(ha, hb, W1a, W1b, b1r, W2c, b2r, block=2048, n_valid=E)
        sums = _sc_segment_sum(m, dst2d, zrows)
        counts = jnp.zeros((TOTAL, 1), jnp.float32).at[nbr_idx].add(1.0)
        h = _gru(sums, counts, h, gik, girk, gru_int_b, block=2000)

        hs = jnp.roll(h, -N_NODES, axis=0)
        m1 = _msg_mlp(h, hs, W1a, W1b, b1r, W2c, b2r, block=2000)
        m2 = _msg_mlp(hs, h, W1a, W1b, b1r, W2c, b2r, block=2000)
        z200 = jnp.zeros((N_NODES, D), jnp.float32)
        tsum = (jnp.concatenate([z200, m1[:TOTAL - N_NODES]], axis=0)
                + jnp.concatenate([m2[:TOTAL - N_NODES], z200], axis=0))
        h = _gru(tsum, tcount, h, gtk, gtrk, gru_temp_b, block=2000)

    W3p = jnp.zeros((512, D), jnp.float32).at[:, :10].set(ro_W3)
    b3p = jnp.full((1, D), -1e30, jnp.float32).at[0, :10].set(ro_b3)
    probs = _readout(h[:N_NODES], ro_W1, ro_b1.reshape(1, H1),
                     ro_W2, ro_b2.reshape(1, 512), W3p, b3p)
    return probs[:, :10]


# speedup vs baseline: 2.0212x; 1.8967x over previous
import functools

import jax
import jax.numpy as jnp
from jax import lax
from jax.experimental import pallas as pl
from jax.experimental.pallas import tpu as pltpu
from jax.experimental.pallas import tpu_sc as plsc

N_NODES = 200
WINDOW = 200
TOTAL = N_NODES * WINDOW
D = 128
T = 2
E = 100000
H1 = 1024

E_PAD = 100352
CHUNK = 128
EPT = E_PAD // 16
NCH = EPT // CHUNK
QC = 16
SEG_PT = TOTAL // 16
BCH = 7
NB = NCH // BCH


def _sc_segment_sum(m, dst2d, zrows):
    mesh = plsc.VectorSubcoreMesh(core_axis_name="c", subcore_axis_name="s")

    @functools.partial(
        pl.kernel, mesh=mesh,
        out_type=jax.ShapeDtypeStruct((TOTAL, D), jnp.float32),
        compiler_params=pltpu.CompilerParams(use_tc_tiling_on_sc=False),
        scratch_types=[
            pltpu.VMEM((NCH, CHUNK), jnp.int32),
            pltpu.VMEM((BCH * CHUNK, QC), jnp.float32),
            pltpu.VMEM((BCH * CHUNK, QC), jnp.float32),
            pltpu.VMEM_SHARED((TOTAL, QC), jnp.float32),
            pltpu.SemaphoreType.DMA,
            pltpu.SemaphoreType.DMA,
            pltpu.SemaphoreType.DMA,
            pltpu.SemaphoreType.DMA,
        ],
    )
    def seg_sum(m_hbm, dst_hbm, z_hbm, out_hbm, idx_v, rows_a, rows_b,
                acc_sh, gs0, gs1, ss0, ss1):
        c = lax.axis_index("c")
        s = lax.axis_index("s")
        bufs = (rows_a, rows_b)
        gsems = (gs0, gs1)
        ssems = (ss0, ss1)
        pltpu.sync_copy(dst_hbm.at[pl.ds(s * NCH, NCH)], idx_v)
        for r in range(4):
            q = c * 4 + r
            col = pl.multiple_of(q * QC, QC)
            pltpu.sync_copy(z_hbm, acc_sh.at[pl.ds(s * SEG_PT, SEG_PT)])
            plsc.subcore_barrier()

            def gather(b):
                return pltpu.async_copy(
                    m_hbm.at[pl.ds(s * EPT + b * BCH * CHUNK, BCH * CHUNK),
                             pl.ds(col, QC)],
                    bufs[b % 2], gsems[b % 2])

            gh = gather(0)
            scat = {0: [], 1: []}
            for b in range(NB):
                cur = b % 2
                nxt = 1 - cur
                if b + 1 < NB:
                    for hd in scat[nxt]:
                        hd.wait()
                    scat[nxt] = []
                    gh_next = gather(b + 1)
                gh.wait()
                for k in range(BCH):
                    scat[cur].append(pltpu.async_copy(
                        bufs[cur].at[pl.ds(k * CHUNK, CHUNK)],
                        acc_sh.at[idx_v.at[b * BCH + k]],
                        ssems[cur], add=True))
                if b + 1 < NB:
                    gh = gh_next
            for side in (0, 1):
                for hd in scat[side]:
                    hd.wait()
            plsc.subcore_barrier()
            pltpu.sync_copy(
                acc_sh.at[pl.ds(s * SEG_PT, SEG_PT)],
                out_hbm.at[pl.ds(s * SEG_PT, SEG_PT), pl.ds(col, QC)])
            if r < 3:
                plsc.subcore_barrier()

    return seg_sum(m, dst2d, zrows)


G_ROWS = 2 * E_PAD
G_RPW = G_ROWS // 32
G_NCH = G_RPW // CHUNK
G_RING = 6


def _sc_gather(h, gidx2d):
    mesh = plsc.VectorSubcoreMesh(core_axis_name="c", subcore_axis_name="s")

    @functools.partial(
        pl.kernel, mesh=mesh,
        out_type=jax.ShapeDtypeStruct((G_ROWS, D), jnp.float32),
        compiler_params=pltpu.CompilerParams(use_tc_tiling_on_sc=False),
        scratch_types=(
            [pltpu.VMEM((G_NCH, CHUNK), jnp.int32)]
            + [pltpu.VMEM((CHUNK, D), jnp.float32)] * G_RING
            + [pltpu.SemaphoreType.DMA] * (2 * G_RING)
        ),
    )
    def gather_k(h_hbm, gidx_hbm, out_hbm, idx_v, *bufsem):
        c = lax.axis_index("c")
        s = lax.axis_index("s")
        w = s * 2 + c
        bufs = bufsem[:G_RING]
        gsem = bufsem[G_RING:2 * G_RING]
        wsem = bufsem[2 * G_RING:]
        pltpu.sync_copy(gidx_hbm.at[pl.ds(w * G_NCH, G_NCH)], idx_v)

        def gth(j, sl):
            return pltpu.async_copy(h_hbm.at[idx_v.at[j]], bufs[sl], gsem[sl])

        def wrb(j, sl):
            return pltpu.async_copy(
                bufs[sl], out_hbm.at[pl.ds(w * G_RPW + j * CHUNK, CHUNK)],
                wsem[sl])

        desc_g = {}
        desc_w = {}
        for j in range(4):
            desc_g[j] = gth(j, j)
        for j in range(G_NCH):
            sl = j % G_RING
            desc_g[sl].wait()
            desc_w[sl] = wrb(j, sl)
            j2 = j + 4
            if j2 < G_NCH:
                s2 = j2 % G_RING
                if s2 in desc_w:
                    desc_w[s2].wait()
                    del desc_w[s2]
                desc_g[s2] = gth(j2, s2)
        for sl in list(desc_w):
            desc_w[sl].wait()

    return gather_k(h, gidx2d).reshape(2, E_PAD, D)


def _msg_mlp(ha, hb, W1a, W1b, b1, W2, b2, *, block, n_valid=None):
    n = ha.shape[0]
    assert n % block == 0
    grid = n // block

    def kern(ha_ref, hb_ref, w1a_ref, w1b_ref, b1_ref, w2_ref, b2_ref, o_ref):
        acc = jnp.dot(ha_ref[...].astype(jnp.bfloat16), w1a_ref[...],
                      preferred_element_type=jnp.float32)
        acc = acc + jnp.dot(hb_ref[...].astype(jnp.bfloat16), w1b_ref[...],
                            preferred_element_type=jnp.float32)
        hid = jnp.maximum(acc + b1_ref[...], 0.0).astype(jnp.bfloat16)
        out = jnp.dot(hid, w2_ref[...], preferred_element_type=jnp.float32) + b2_ref[...]
        out = jnp.maximum(out, 0.0)
        if n_valid is not None:
            row = (lax.broadcasted_iota(jnp.int32, (block, 1), 0)
                   + pl.program_id(0) * block)
            out = jnp.where(row < n_valid, out, 0.0)
        o_ref[...] = out

    return pl.pallas_call(
        kern,
        grid=(grid,),
        in_specs=[
            pl.BlockSpec((block, D), lambda i: (i, 0)),
            pl.BlockSpec((block, D), lambda i: (i, 0)),
            pl.BlockSpec((D, H1), lambda i: (0, 0)),
            pl.BlockSpec((D, H1), lambda i: (0, 0)),
            pl.BlockSpec((1, H1), lambda i: (0, 0)),
            pl.BlockSpec((H1, D), lambda i: (0, 0)),
            pl.BlockSpec((1, D), lambda i: (0, 0)),
        ],
        out_specs=pl.BlockSpec((block, D), lambda i: (i, 0)),
        out_shape=jax.ShapeDtypeStruct((n, D), jnp.float32),
    )(ha, hb, W1a, W1b, b1, W2, b2)


def _msg_mlp_packed(gath3, W1a, W1b, b1, W2, b2, *, block, n_valid):
    n = gath3.shape[1]
    assert n % block == 0
    grid = n // block

    def kern(ha_ref, hb_ref, w1a_ref, w1b_ref, b1_ref, w2_ref, b2_ref, o_ref):
        acc = jnp.dot(ha_ref[0].astype(jnp.bfloat16), w1a_ref[...],
                      preferred_element_type=jnp.float32)
        acc = acc + jnp.dot(hb_ref[0].astype(jnp.bfloat16), w1b_ref[...],
                            preferred_element_type=jnp.float32)
        hid = jnp.maximum(acc + b1_ref[...], 0.0).astype(jnp.bfloat16)
        out = jnp.dot(hid, w2_ref[...], preferred_element_type=jnp.float32) + b2_ref[...]
        out = jnp.maximum(out, 0.0)
        row = (lax.broadcasted_iota(jnp.int32, (block, 1), 0)
               + pl.program_id(0) * block)
        o_ref[...] = jnp.where(row < n_valid, out, 0.0)

    return pl.pallas_call(
        kern,
        grid=(grid,),
        in_specs=[
            pl.BlockSpec((1, block, D), lambda i: (0, i, 0)),
            pl.BlockSpec((1, block, D), lambda i: (1, i, 0)),
            pl.BlockSpec((D, H1), lambda i: (0, 0)),
            pl.BlockSpec((D, H1), lambda i: (0, 0)),
            pl.BlockSpec((1, H1), lambda i: (0, 0)),
            pl.BlockSpec((H1, D), lambda i: (0, 0)),
            pl.BlockSpec((1, D), lambda i: (0, 0)),
        ],
        out_specs=pl.BlockSpec((block, D), lambda i: (i, 0)),
        out_shape=jax.ShapeDtypeStruct((n, D), jnp.float32),
    )(gath3, gath3, W1a, W1b, b1, W2, b2)


def _gru(sums, counts, h, k, rk, b, *, block):
    n = h.shape[0]
    assert n % block == 0
    grid = n // block

    def kern(s_ref, c_ref, h_ref, k_ref, rk_ref, b_ref, o_ref):
        c = c_ref[...]
        x = jnp.where(c > 0.0, s_ref[...] / jnp.maximum(c, 1.0), 0.0)
        hv = h_ref[...]
        mx = jnp.dot(x.astype(jnp.bfloat16), k_ref[...],
                     preferred_element_type=jnp.float32) + b_ref[0:1, :]
        mh = jnp.dot(hv.astype(jnp.bfloat16), rk_ref[...],
                     preferred_element_type=jnp.float32) + b_ref[1:2, :]
        z = jax.nn.sigmoid(mx[:, :D] + mh[:, :D])
        r = jax.nn.sigmoid(mx[:, D:2 * D] + mh[:, D:2 * D])
        cand = jnp.tanh(mx[:, 2 * D:] + r * mh[:, 2 * D:])
        o_ref[...] = z * hv + (1.0 - z) * cand

    return pl.pallas_call(
        kern,
        grid=(grid,),
        in_specs=[
            pl.BlockSpec((block, D), lambda i: (i, 0)),
            pl.BlockSpec((block, 1), lambda i: (i, 0)),
            pl.BlockSpec((block, D), lambda i: (i, 0)),
            pl.BlockSpec((D, 3 * D), lambda i: (0, 0)),
            pl.BlockSpec((D, 3 * D), lambda i: (0, 0)),
            pl.BlockSpec((2, 3 * D), lambda i: (0, 0)),
        ],
        out_specs=pl.BlockSpec((block, D), lambda i: (i, 0)),
        out_shape=jax.ShapeDtypeStruct((n, D), jnp.float32),
    )(sums, counts, h, k, rk, b)


def _readout(x, W1, b1, W2, b2, W3p, b3p):
    def kern(x_ref, w1_ref, b1_ref, w2_ref, b2_ref, w3_ref, b3_ref, o_ref):
        a = jnp.maximum(
            jnp.dot(x_ref[...], w1_ref[...], preferred_element_type=jnp.float32)
            + b1_ref[...], 0.0)
        a = jnp.maximum(
            jnp.dot(a, w2_ref[...], preferred_element_type=jnp.float32)
            + b2_ref[...], 0.0)
        lg = jnp.dot(a, w3_ref[...], preferred_element_type=jnp.float32) + b3_ref[...]
        mx = jnp.max(lg, axis=-1, keepdims=True)
        ex = jnp.exp(lg - mx)
        o_ref[...] = ex / jnp.sum(ex, axis=-1, keepdims=True)

    return pl.pallas_call(
        kern,
        out_shape=jax.ShapeDtypeStruct((N_NODES, D), jnp.float32),
    )(x, W1, b1, W2, b2, W3p, b3p)


def kernel(int_edges, nodes, embed, msg_W1, msg_b1, msg_W2, msg_b2,
           gru_int_k, gru_int_rk, gru_int_b, gru_temp_k, gru_temp_rk,
           gru_temp_b, ro_W1, ro_b1, ro_W2, ro_b2, ro_W3, ro_b3):
    node_idx = int_edges[:, 0] * N_NODES + int_edges[:, 1]
    nbr_idx = int_edges[:, 0] * N_NODES + int_edges[:, 2]
    ipad = jnp.zeros((E_PAD - E,), jnp.int32)
    node_idx_p = jnp.concatenate([node_idx, ipad])
    nbr_idx_p = jnp.concatenate([nbr_idx, ipad])
    dst2d = nbr_idx_p.reshape(E_PAD // CHUNK, CHUNK)
    gidx2d = jnp.concatenate([node_idx_p, nbr_idx_p]).reshape(
        G_ROWS // CHUNK, CHUNK)
    zrows = jnp.zeros((SEG_PT, QC), jnp.float32)

    h = jnp.repeat(embed[nodes], WINDOW, axis=0)

    W1a = msg_W1[:D].astype(jnp.bfloat16)
    W1b = msg_W1[D:].astype(jnp.bfloat16)
    b1r = msg_b1.reshape(1, H1)
    b2r = msg_b2.reshape(1, D)
    W2c = msg_W2.astype(jnp.bfloat16)
    gik = gru_int_k.astype(jnp.bfloat16)
    girk = gru_int_rk.astype(jnp.bfloat16)
    gtk = gru_temp_k.astype(jnp.bfloat16)
    gtrk = gru_temp_rk.astype(jnp.bfloat16)

    io = jnp.arange(TOTAL, dtype=jnp.int32)
    tcount = jnp.where((io >= N_NODES) & (io < TOTAL - N_NODES), 2.0, 1.0)[:, None]

    for _ in range(T):
        gath3 = _sc_gather(h, gidx2d)
        m = _msg_mlp_packed(gath3, W1a, W1b, b1r, W2c, b2r, block=2048, n_valid=E)
        sums = _sc_segment_sum(m, dst2d, zrows)
        counts = jnp.zeros((TOTAL, 1), jnp.float32).at[nbr_idx].add(1.0)
        h = _gru(sums, counts, h, gik, girk, gru_int_b, block=2000)

        hs = jnp.roll(h, -N_NODES, axis=0)
        m1 = _msg_mlp(h, hs, W1a, W1b, b1r, W2c, b2r, block=2000)
        m2 = _msg_mlp(hs, h, W1a, W1b, b1r, W2c, b2r, block=2000)
        z200 = jnp.zeros((N_NODES, D), jnp.float32)
        tsum = (jnp.concatenate([z200, m1[:TOTAL - N_NODES]], axis=0)
                + jnp.concatenate([m2[:TOTAL - N_NODES], z200], axis=0))
        h = _gru(tsum, tcount, h, gtk, gtrk, gru_temp_b, block=2000)

    W3p = jnp.zeros((512, D), jnp.float32).at[:, :10].set(ro_W3)
    b3p = jnp.full((1, D), -1e30, jnp.float32).at[0, :10].set(ro_b3)
    probs = _readout(h[:N_NODES], ro_W1, ro_b1.reshape(1, H1),
                     ro_W2, ro_b2.reshape(1, 512), W3p, b3p)
    return probs[:, :10]
